# baseline (speedup 1.0000x reference)
import functools

import jax
import jax.numpy as jnp
from jax import lax
from jax.experimental import pallas as pl
from jax.experimental.pallas import tpu as pltpu
from jax.experimental.pallas import tpu_sc as plsc

NC = 2
NS = 16
NW = NC * NS
CHUNK = 128


def _sc_segment_sum(h_pad, src_w, dst_w, zeros_d, n_pad, d):
  k = src_w.shape[1]
  rows_per_tile = n_pad // NS
  mesh = plsc.VectorSubcoreMesh(core_axis_name="c", subcore_axis_name="s")

  @functools.partial(
      pl.kernel,
      out_type=jax.ShapeDtypeStruct((NC, n_pad, d), jnp.float32),
      mesh=mesh,
      scratch_types=[
          pltpu.VMEM((CHUNK,), jnp.int32),
          pltpu.VMEM((CHUNK,), jnp.int32),
          pltpu.VMEM((k, CHUNK), jnp.int32),
          pltpu.VMEM((CHUNK, d), jnp.float32),
          pltpu.VMEM((CHUNK, d), jnp.float32),
          pltpu.VMEM_SHARED((n_pad, d), jnp.float32),
          pltpu.SemaphoreType.DMA,
          pltpu.SemaphoreType.DMA,
          pltpu.SemaphoreType.DMA,
          pltpu.SemaphoreType.DMA,
          pltpu.SemaphoreType.DMA,
          pltpu.SemaphoreType.DMA,
      ],
  )
  def kfn(h_hbm, src_hbm, dst_hbm, z_hbm, out_hbm,
          sidx0, sidx1, dst_v, buf0, buf1, acc,
          gsem0, gsem1, ssem0, ssem1, isem0, isem1):
    cid = lax.axis_index("c")
    sid = lax.axis_index("s")
    wid = sid * NC + cid
    bufs = (buf0, buf1)
    sidxs = (sidx0, sidx1)
    gsems = (gsem0, gsem1)
    ssems = (ssem0, ssem1)
    isems = (isem0, isem1)
    pltpu.sync_copy(src_hbm.at[wid, 0], sidx0)
    pltpu.async_copy(h_hbm.at[sidx0], buf0, gsem0)
    pltpu.async_copy(src_hbm.at[wid, 1], sidx1, isem1)
    pltpu.sync_copy(dst_hbm.at[wid], dst_v)
    pltpu.sync_copy(z_hbm, buf1)
    r0 = sid * rows_per_tile
    for t in range(rows_per_tile // CHUNK):
      pltpu.sync_copy(buf1, acc.at[pl.ds(r0 + t * CHUNK, CHUNK)])
    plsc.subcore_barrier()

    def pair(jq, carry):
      for p in range(2):
        j = jq * 2 + p
        nxt = j + 1
        pltpu.make_async_copy(h_hbm.at[sidxs[p]], bufs[p], gsems[p]).wait()

        @pl.when(j + 2 < k)
        def _():
          pltpu.async_copy(src_hbm.at[wid, j + 2], sidxs[p], isems[p])

        pltpu.async_copy(bufs[p], acc.at[dst_v.at[j]], ssems[p], add=True)

        @pl.when(nxt < k)
        def _():
          @pl.when(j >= 1)
          def _():
            pltpu.make_async_copy(bufs[1 - p], acc.at[dst_v.at[j - 1]],
                                  ssems[1 - p]).wait()
          pltpu.make_async_copy(src_hbm.at[wid, nxt], sidxs[1 - p],
                                isems[1 - p]).wait()
          pltpu.async_copy(h_hbm.at[sidxs[1 - p]], bufs[1 - p], gsems[1 - p])
      return carry

    lax.fori_loop(0, k // 2, pair, 0)
    pltpu.make_async_copy(bufs[0], acc.at[dst_v.at[k - 2]], ssems[0]).wait()
    pltpu.make_async_copy(bufs[1], acc.at[dst_v.at[k - 1]], ssems[1]).wait()
    plsc.subcore_barrier()
    pltpu.sync_copy(acc.at[pl.ds(r0, rows_per_tile)],
                    out_hbm.at[cid, pl.ds(r0, rows_per_tile)])

  return kfn(h_pad, src_w, dst_w, zeros_d)


def _sc_degree(dst_w, zeros_d, ones_d, n_pad, d):
  k = dst_w.shape[1]
  rows_per_tile = n_pad // NS
  mesh = plsc.VectorSubcoreMesh(core_axis_name="c", subcore_axis_name="s")

  @functools.partial(
      pl.kernel,
      out_type=jax.ShapeDtypeStruct((NC, n_pad, d), jnp.float32),
      mesh=mesh,
      scratch_types=[
          pltpu.VMEM((k, CHUNK), jnp.int32),
          pltpu.VMEM((CHUNK, d), jnp.float32),
          pltpu.VMEM_SHARED((n_pad, d), jnp.float32),
          pltpu.SemaphoreType.DMA,
      ],
  )
  def kfn(dst_hbm, z_hbm, o_hbm, out_hbm, dst_v, buf, acc, sem):
    cid = lax.axis_index("c")
    sid = lax.axis_index("s")
    wid = sid * NC + cid
    pltpu.sync_copy(dst_hbm.at[wid], dst_v)
    pltpu.sync_copy(z_hbm, buf)
    r0 = sid * rows_per_tile
    for t in range(rows_per_tile // CHUNK):
      pltpu.sync_copy(buf, acc.at[pl.ds(r0 + t * CHUNK, CHUNK)])
    plsc.subcore_barrier()
    pltpu.sync_copy(o_hbm, buf)

    def fire(j, carry):
      pltpu.async_copy(buf, acc.at[dst_v.at[j]], sem, add=True)
      return carry

    lax.fori_loop(0, k, fire, 0)

    def drain(j, carry):
      pltpu.make_async_copy(buf, acc.at[dst_v.at[j]], sem).wait()
      return carry

    lax.fori_loop(0, k, drain, 0)
    plsc.subcore_barrier()
    pltpu.sync_copy(acc.at[pl.ds(r0, rows_per_tile)],
                    out_hbm.at[cid, pl.ds(r0, rows_per_tile)])

  return kfn(dst_w, zeros_d, ones_d)


def _tc_embed(nt2, emb_pad, n_pad, d, g_pad, bn=512):
  def body(nt_ref, emb_ref, out_ref):
    idx = nt_ref[...]
    oh = (idx == lax.broadcasted_iota(jnp.int32, (bn, g_pad), 1)
          ).astype(jnp.float32)
    out_ref[...] = jnp.dot(oh, emb_ref[...],
                           preferred_element_type=jnp.float32)

  return pl.pallas_call(
      body,
      grid=(n_pad // bn,),
      in_specs=[
          pl.BlockSpec((bn, 1), lambda i: (i, 0)),
          pl.BlockSpec((g_pad, d), lambda i: (0, 0)),
      ],
      out_specs=pl.BlockSpec((bn, d), lambda i: (i, 0)),
      out_shape=jax.ShapeDtypeStruct((n_pad, d), jnp.float32),
  )(nt2, emb_pad)


def _tc_update(h, p0, p1, dg0, dg1, ws, wn, bias, n_pad, d, bn=512):
  def body(h_ref, p0_ref, p1_ref, d0_ref, d1_ref, ws_ref, wn_ref, b_ref,
           out_ref):
    deg = jnp.maximum(d0_ref[...][:, :1] + d1_ref[...][:, :1], 1.0)
    agg = (p0_ref[...] + p1_ref[...]) / deg
    out_ref[...] = jnp.maximum(
        jnp.dot(h_ref[...], ws_ref[...], preferred_element_type=jnp.float32)
        + jnp.dot(agg, wn_ref[...], preferred_element_type=jnp.float32)
        + b_ref[...], 0.0)

  full = lambda i: (0, 0)
  row = lambda i: (i, 0)
  return pl.pallas_call(
      body,
      grid=(n_pad // bn,),
      in_specs=[
          pl.BlockSpec((bn, d), row),
          pl.BlockSpec((bn, d), row),
          pl.BlockSpec((bn, d), row),
          pl.BlockSpec((bn, 16), row),
          pl.BlockSpec((bn, 16), row),
          pl.BlockSpec((d, d), full),
          pl.BlockSpec((d, d), full),
          pl.BlockSpec((1, d), full),
      ],
      out_specs=pl.BlockSpec((bn, d), row),
      out_shape=jax.ShapeDtypeStruct((n_pad, d), jnp.float32),
  )(h, p0, p1, dg0, dg1, ws, wn, bias)


def _tc_heads(h, aw1, ab1, aw2, ab2, cw1, cb1, cw2r, cb2, n_pad, d, ah, act,
              ch, bn=512):
  def body(h_ref, aw1_ref, ab1_ref, aw2_ref, ab2_ref, cw1_ref, cb1_ref,
           cw2_ref, cb2_ref, v_ref, lg_ref):
    hh = h_ref[...]
    chid = jnp.maximum(
        jnp.dot(hh, cw1_ref[...], preferred_element_type=jnp.float32)
        + cb1_ref[...], 0.0)
    v_ref[...] = (jnp.sum(chid * cw2_ref[...], axis=1, keepdims=True)
                  + cb2_ref[0, 0])
    ahid = jnp.maximum(
        jnp.dot(hh, aw1_ref[...], preferred_element_type=jnp.float32)
        + ab1_ref[...], 0.0)
    lg_ref[...] = (jnp.dot(ahid, aw2_ref[...],
                           preferred_element_type=jnp.float32) + ab2_ref[...])

  full = lambda i: (0, 0)
  row = lambda i: (i, 0)
  return pl.pallas_call(
      body,
      grid=(n_pad // bn,),
      in_specs=[
          pl.BlockSpec((bn, d), row),
          pl.BlockSpec((d, ah), full),
          pl.BlockSpec((1, ah), full),
          pl.BlockSpec((ah, act), full),
          pl.BlockSpec((1, act), full),
          pl.BlockSpec((d, ch), full),
          pl.BlockSpec((1, ch), full),
          pl.BlockSpec((1, ch), full),
          pl.BlockSpec((1, 1), full),
      ],
      out_specs=[
          pl.BlockSpec((bn, 1), row),
          pl.BlockSpec((bn, act), row),
      ],
      out_shape=[
          jax.ShapeDtypeStruct((n_pad, 1), jnp.float32),
          jax.ShapeDtypeStruct((n_pad, act), jnp.float32),
      ],
  )(h, aw1, ab1, aw2, ab2, cw1, cb1, cw2r, cb2)


def _prep_edges(src, dst, n, parts):
  e = src.shape[0]
  per_w = -(-e // parts)
  k = -(-per_w // CHUNK)
  k = ((k + 3) // 4) * 4
  pad_flat = parts * per_w - e
  src = jnp.concatenate([src, jnp.full((pad_flat,), n, jnp.int32)])
  dst = jnp.concatenate([dst, jnp.full((pad_flat,), n, jnp.int32)])
  src = src.reshape(parts, per_w)
  dst = dst.reshape(parts, per_w)
  pad_w = k * CHUNK - per_w
  src_w = jnp.concatenate(
      [src, jnp.full((parts, pad_w), n, jnp.int32)],
      axis=1).reshape(parts, k, CHUNK)
  dst_w = jnp.concatenate(
      [dst, jnp.full((parts, pad_w), n, jnp.int32)],
      axis=1).reshape(parts, k, CHUNK)
  return src_w, dst_w


def kernel(node_type, edge_index, emb, W_self, W_neigh, b,
           aW1, ab1, aW2, ab2, cW1, cb1, cW2, cb2):
  n = node_type.shape[0]
  e = edge_index.shape[1]
  g, d = emb.shape
  num_layers = W_self.shape[0]
  ah = aW1.shape[1]
  act = aW2.shape[1]
  ch = cW1.shape[1]

  stripe = NS * CHUNK
  n_pad = ((n + 1 + stripe - 1) // stripe) * stripe
  g_pad = ((g + 7) // 8) * 8

  nt = node_type.astype(jnp.int32)
  src = edge_index[0].astype(jnp.int32)
  dst = edge_index[1].astype(jnp.int32)
  src_w, dst_w = _prep_edges(src, dst, n, NW)

  zeros_d = jnp.zeros((CHUNK, d), jnp.float32)
  ones_d = jnp.ones((CHUNK, d), jnp.float32)

  nt2 = jnp.pad(nt, (0, n_pad - n)).reshape(n_pad, 1)
  emb_pad = jnp.pad(emb, ((0, g_pad - g), (0, 0)))

  h = _tc_embed(nt2, emb_pad, n_pad, d, g_pad)
  degp = _sc_degree(dst_w, zeros_d, ones_d, n_pad, d)
  dg0 = degp[0, :, :16]
  dg1 = degp[1, :, :16]

  for l in range(num_layers):
    part = _sc_segment_sum(h, src_w, dst_w, zeros_d, n_pad, d)
    h = _tc_update(h, part[0], part[1], dg0, dg1,
                   W_self[l], W_neigh[l], b[l].reshape(1, d), n_pad, d)

  vs, lg = _tc_heads(h, aW1, ab1.reshape(1, ah), aW2, ab2.reshape(1, act),
                     cW1, cb1.reshape(1, ch), cW2.reshape(1, ch),
                     cb2.reshape(1, 1), n_pad, d, ah, act, ch)
  return jnp.concatenate([vs[:n], lg[:n]], axis=1)


# speedup vs baseline: 1.0007x; 1.0007x over previous
import functools

import jax
import jax.numpy as jnp
from jax import lax
from jax.experimental import pallas as pl
from jax.experimental.pallas import tpu as pltpu
from jax.experimental.pallas import tpu_sc as plsc

NC = 2
NS = 16
NW = NC * NS
CHUNK = 128


def _sc_segment_sum(h_pad, src_w, dst_w, zeros_d, n_pad, d):
  k = src_w.shape[1]
  rows_per_tile = n_pad // NS
  mesh = plsc.VectorSubcoreMesh(core_axis_name="c", subcore_axis_name="s")

  @functools.partial(
      pl.kernel,
      out_type=jax.ShapeDtypeStruct((NC, n_pad, d), jnp.float32),
      mesh=mesh,
      scratch_types=[
          pltpu.VMEM((CHUNK,), jnp.int32),
          pltpu.VMEM((CHUNK,), jnp.int32),
          pltpu.VMEM((k, CHUNK), jnp.int32),
          pltpu.VMEM((CHUNK, d), jnp.float32),
          pltpu.VMEM((CHUNK, d), jnp.float32),
          pltpu.VMEM_SHARED((n_pad, d), jnp.float32),
          pltpu.SemaphoreType.DMA,
          pltpu.SemaphoreType.DMA,
          pltpu.SemaphoreType.DMA,
          pltpu.SemaphoreType.DMA,
          pltpu.SemaphoreType.DMA,
          pltpu.SemaphoreType.DMA,
      ],
  )
  def kfn(h_hbm, src_hbm, dst_hbm, z_hbm, out_hbm,
          sidx0, sidx1, dst_v, buf0, buf1, acc,
          gsem0, gsem1, ssem0, ssem1, isem0, isem1):
    cid = lax.axis_index("c")
    sid = lax.axis_index("s")
    wid = sid * NC + cid
    bufs = (buf0, buf1)
    sidxs = (sidx0, sidx1)
    gsems = (gsem0, gsem1)
    ssems = (ssem0, ssem1)
    isems = (isem0, isem1)
    half = CHUNK // 2

    def gfire(idxref, bufref, sem):
      pltpu.async_copy(h_hbm.at[idxref.at[pl.ds(0, half)]],
                       bufref.at[pl.ds(0, half)], sem)
      pltpu.async_copy(h_hbm.at[idxref.at[pl.ds(half, half)]],
                       bufref.at[pl.ds(half, half)], sem)

    def gwait(idxref, bufref, sem):
      pltpu.make_async_copy(h_hbm.at[idxref.at[pl.ds(0, half)]],
                            bufref.at[pl.ds(0, half)], sem).wait()
      pltpu.make_async_copy(h_hbm.at[idxref.at[pl.ds(half, half)]],
                            bufref.at[pl.ds(half, half)], sem).wait()

    pltpu.sync_copy(src_hbm.at[wid, 0], sidx0)
    gfire(sidx0, buf0, gsem0)
    pltpu.async_copy(src_hbm.at[wid, 1], sidx1, isem1)
    pltpu.sync_copy(dst_hbm.at[wid], dst_v)
    pltpu.sync_copy(z_hbm, buf1)
    r0 = sid * rows_per_tile
    for t in range(rows_per_tile // CHUNK):
      pltpu.sync_copy(buf1, acc.at[pl.ds(r0 + t * CHUNK, CHUNK)])
    plsc.subcore_barrier()

    def pair(jq, carry):
      for p in range(2):
        j = jq * 2 + p
        nxt = j + 1
        gwait(sidxs[p], bufs[p], gsems[p])

        @pl.when(j + 2 < k)
        def _():
          pltpu.async_copy(src_hbm.at[wid, j + 2], sidxs[p], isems[p])

        pltpu.async_copy(bufs[p], acc.at[dst_v.at[j]], ssems[p], add=True)

        @pl.when(nxt < k)
        def _():
          @pl.when(j >= 1)
          def _():
            pltpu.make_async_copy(bufs[1 - p], acc.at[dst_v.at[j - 1]],
                                  ssems[1 - p]).wait()
          pltpu.make_async_copy(src_hbm.at[wid, nxt], sidxs[1 - p],
                                isems[1 - p]).wait()
          gfire(sidxs[1 - p], bufs[1 - p], gsems[1 - p])
      return carry

    lax.fori_loop(0, k // 2, pair, 0)
    pltpu.make_async_copy(bufs[0], acc.at[dst_v.at[k - 2]], ssems[0]).wait()
    pltpu.make_async_copy(bufs[1], acc.at[dst_v.at[k - 1]], ssems[1]).wait()
    plsc.subcore_barrier()
    pltpu.sync_copy(acc.at[pl.ds(r0, rows_per_tile)],
                    out_hbm.at[cid, pl.ds(r0, rows_per_tile)])

  return kfn(h_pad, src_w, dst_w, zeros_d)


def _sc_degree(dst_w, zeros_d, ones_d, n_pad, d):
  k = dst_w.shape[1]
  rows_per_tile = n_pad // NS
  mesh = plsc.VectorSubcoreMesh(core_axis_name="c", subcore_axis_name="s")

  @functools.partial(
      pl.kernel,
      out_type=jax.ShapeDtypeStruct((NC, n_pad, d), jnp.float32),
      mesh=mesh,
      scratch_types=[
          pltpu.VMEM((k, CHUNK), jnp.int32),
          pltpu.VMEM((CHUNK, d), jnp.float32),
          pltpu.VMEM_SHARED((n_pad, d), jnp.float32),
          pltpu.SemaphoreType.DMA,
      ],
  )
  def kfn(dst_hbm, z_hbm, o_hbm, out_hbm, dst_v, buf, acc, sem):
    cid = lax.axis_index("c")
    sid = lax.axis_index("s")
    wid = sid * NC + cid
    pltpu.sync_copy(dst_hbm.at[wid], dst_v)
    pltpu.sync_copy(z_hbm, buf)
    r0 = sid * rows_per_tile
    for t in range(rows_per_tile // CHUNK):
      pltpu.sync_copy(buf, acc.at[pl.ds(r0 + t * CHUNK, CHUNK)])
    plsc.subcore_barrier()
    pltpu.sync_copy(o_hbm, buf)

    def fire(j, carry):
      pltpu.async_copy(buf, acc.at[dst_v.at[j]], sem, add=True)
      return carry

    lax.fori_loop(0, k, fire, 0)

    def drain(j, carry):
      pltpu.make_async_copy(buf, acc.at[dst_v.at[j]], sem).wait()
      return carry

    lax.fori_loop(0, k, drain, 0)
    plsc.subcore_barrier()
    pltpu.sync_copy(acc.at[pl.ds(r0, rows_per_tile)],
                    out_hbm.at[cid, pl.ds(r0, rows_per_tile)])

  return kfn(dst_w, zeros_d, ones_d)


def _tc_embed(nt2, emb_pad, n_pad, d, g_pad, bn=512):
  def body(nt_ref, emb_ref, out_ref):
    idx = nt_ref[...]
    oh = (idx == lax.broadcasted_iota(jnp.int32, (bn, g_pad), 1)
          ).astype(jnp.float32)
    out_ref[...] = jnp.dot(oh, emb_ref[...],
                           preferred_element_type=jnp.float32)

  return pl.pallas_call(
      body,
      grid=(n_pad // bn,),
      in_specs=[
          pl.BlockSpec((bn, 1), lambda i: (i, 0)),
          pl.BlockSpec((g_pad, d), lambda i: (0, 0)),
      ],
      out_specs=pl.BlockSpec((bn, d), lambda i: (i, 0)),
      out_shape=jax.ShapeDtypeStruct((n_pad, d), jnp.float32),
  )(nt2, emb_pad)


def _tc_update(h, p0, p1, dg0, dg1, ws, wn, bias, n_pad, d, bn=512):
  def body(h_ref, p0_ref, p1_ref, d0_ref, d1_ref, ws_ref, wn_ref, b_ref,
           out_ref):
    deg = jnp.maximum(d0_ref[...][:, :1] + d1_ref[...][:, :1], 1.0)
    agg = (p0_ref[...] + p1_ref[...]) / deg
    out_ref[...] = jnp.maximum(
        jnp.dot(h_ref[...], ws_ref[...], preferred_element_type=jnp.float32)
        + jnp.dot(agg, wn_ref[...], preferred_element_type=jnp.float32)
        + b_ref[...], 0.0)

  full = lambda i: (0, 0)
  row = lambda i: (i, 0)
  return pl.pallas_call(
      body,
      grid=(n_pad // bn,),
      in_specs=[
          pl.BlockSpec((bn, d), row),
          pl.BlockSpec((bn, d), row),
          pl.BlockSpec((bn, d), row),
          pl.BlockSpec((bn, 16), row),
          pl.BlockSpec((bn, 16), row),
          pl.BlockSpec((d, d), full),
          pl.BlockSpec((d, d), full),
          pl.BlockSpec((1, d), full),
      ],
      out_specs=pl.BlockSpec((bn, d), row),
      out_shape=jax.ShapeDtypeStruct((n_pad, d), jnp.float32),
  )(h, p0, p1, dg0, dg1, ws, wn, bias)


def _tc_heads(h, aw1, ab1, aw2, ab2, cw1, cb1, cw2r, cb2, n_pad, d, ah, act,
              ch, bn=512):
  def body(h_ref, aw1_ref, ab1_ref, aw2_ref, ab2_ref, cw1_ref, cb1_ref,
           cw2_ref, cb2_ref, v_ref, lg_ref):
    hh = h_ref[...]
    chid = jnp.maximum(
        jnp.dot(hh, cw1_ref[...], preferred_element_type=jnp.float32)
        + cb1_ref[...], 0.0)
    v_ref[...] = (jnp.sum(chid * cw2_ref[...], axis=1, keepdims=True)
                  + cb2_ref[0, 0])
    ahid = jnp.maximum(
        jnp.dot(hh, aw1_ref[...], preferred_element_type=jnp.float32)
        + ab1_ref[...], 0.0)
    lg_ref[...] = (jnp.dot(ahid, aw2_ref[...],
                           preferred_element_type=jnp.float32) + ab2_ref[...])

  full = lambda i: (0, 0)
  row = lambda i: (i, 0)
  return pl.pallas_call(
      body,
      grid=(n_pad // bn,),
      in_specs=[
          pl.BlockSpec((bn, d), row),
          pl.BlockSpec((d, ah), full),
          pl.BlockSpec((1, ah), full),
          pl.BlockSpec((ah, act), full),
          pl.BlockSpec((1, act), full),
          pl.BlockSpec((d, ch), full),
          pl.BlockSpec((1, ch), full),
          pl.BlockSpec((1, ch), full),
          pl.BlockSpec((1, 1), full),
      ],
      out_specs=[
          pl.BlockSpec((bn, 1), row),
          pl.BlockSpec((bn, act), row),
      ],
      out_shape=[
          jax.ShapeDtypeStruct((n_pad, 1), jnp.float32),
          jax.ShapeDtypeStruct((n_pad, act), jnp.float32),
      ],
  )(h, aw1, ab1, aw2, ab2, cw1, cb1, cw2r, cb2)


def _prep_edges(src, dst, n, parts):
  e = src.shape[0]
  per_w = -(-e // parts)
  k = -(-per_w // CHUNK)
  k = ((k + 3) // 4) * 4
  pad_flat = parts * per_w - e
  src = jnp.concatenate([src, jnp.full((pad_flat,), n, jnp.int32)])
  dst = jnp.concatenate([dst, jnp.full((pad_flat,), n, jnp.int32)])
  src = src.reshape(parts, per_w)
  dst = dst.reshape(parts, per_w)
  pad_w = k * CHUNK - per_w
  src_w = jnp.concatenate(
      [src, jnp.full((parts, pad_w), n, jnp.int32)],
      axis=1).reshape(parts, k, CHUNK)
  dst_w = jnp.concatenate(
      [dst, jnp.full((parts, pad_w), n, jnp.int32)],
      axis=1).reshape(parts, k, CHUNK)
  return src_w, dst_w


def kernel(node_type, edge_index, emb, W_self, W_neigh, b,
           aW1, ab1, aW2, ab2, cW1, cb1, cW2, cb2):
  n = node_type.shape[0]
  e = edge_index.shape[1]
  g, d = emb.shape
  num_layers = W_self.shape[0]
  ah = aW1.shape[1]
  act = aW2.shape[1]
  ch = cW1.shape[1]

  stripe = NS * CHUNK
  n_pad = ((n + 1 + stripe - 1) // stripe) * stripe
  g_pad = ((g + 7) // 8) * 8

  nt = node_type.astype(jnp.int32)
  src = edge_index[0].astype(jnp.int32)
  dst = edge_index[1].astype(jnp.int32)
  src_w, dst_w = _prep_edges(src, dst, n, NW)

  zeros_d = jnp.zeros((CHUNK, d), jnp.float32)
  ones_d = jnp.ones((CHUNK, d), jnp.float32)

  nt2 = jnp.pad(nt, (0, n_pad - n)).reshape(n_pad, 1)
  emb_pad = jnp.pad(emb, ((0, g_pad - g), (0, 0)))

  h = _tc_embed(nt2, emb_pad, n_pad, d, g_pad)
  degp = _sc_degree(dst_w, zeros_d, ones_d, n_pad, d)
  dg0 = degp[0, :, :16]
  dg1 = degp[1, :, :16]

  for l in range(num_layers):
    part = _sc_segment_sum(h, src_w, dst_w, zeros_d, n_pad, d)
    h = _tc_update(h, part[0], part[1], dg0, dg1,
                   W_self[l], W_neigh[l], b[l].reshape(1, d), n_pad, d)

  vs, lg = _tc_heads(h, aW1, ab1.reshape(1, ah), aW2, ab2.reshape(1, act),
                     cW1, cb1.reshape(1, ch), cW2.reshape(1, ch),
                     cb2.reshape(1, 1), n_pad, d, ah, act, ch)
  return jnp.concatenate([vs[:n], lg[:n]], axis=1)
